# baseline (device time: 258970 ns/iter reference)
import jax
import jax.numpy as jnp
from jax import lax
from jax.experimental import pallas as pl
from jax.experimental.pallas import tpu as pltpu

B, SQ, H, D = 4, 32, 8, 128
BK = 512


def kernel(Q, K, V):
    skv = K.shape[1]
    nsteps = skv // BK
    scale = D ** -0.5

    Q = Q.reshape(B, SQ, H * D)
    K = K.reshape(B, skv, H * D)
    V = V.reshape(B, skv, H * D)

    def body(q_ref, k_ref, v_ref, o_ref,
             acc, m, l, acc_tx, acc_rx, stats_tx, stats_rx,
             o_send, o_recv, s_send, s_recv):
        i = pl.program_id(0)

        @pl.when(i == 0)
        def _init():
            m[...] = jnp.full(m.shape, -jnp.inf, jnp.float32)
            l[...] = jnp.zeros(l.shape, jnp.float32)
            acc[...] = jnp.zeros(acc.shape, jnp.float32)

        for bb in range(B):
            kb = k_ref[bb].astype(jnp.bfloat16)
            vb = v_ref[bb].astype(jnp.bfloat16)
            qb = q_ref[bb].astype(jnp.bfloat16)
            for hh in range(H):
                d0 = hh * D
                q = qb[:, d0:d0 + D]
                k = kb[:, d0:d0 + D]
                v = vb[:, d0:d0 + D]
                s = lax.dot_general(
                    q, k, (((1,), (1,)), ((), ())),
                    preferred_element_type=jnp.float32) * scale
                m_cur = jnp.max(s, axis=-1, keepdims=True)
                m_old = m[bb, hh]
                m_new = jnp.maximum(m_old, m_cur)
                alpha = jnp.exp(m_old - m_new)
                p = jnp.exp(s - m_new)
                l[bb, hh] = l[bb, hh] * alpha + jnp.sum(p, axis=-1, keepdims=True)
                pv = lax.dot_general(
                    p.astype(jnp.bfloat16), v, (((1,), (0,)), ((), ())),
                    preferred_element_type=jnp.float32)
                acc[hh, bb] = acc[hh, bb] * alpha + pv
                m[bb, hh] = m_new

        @pl.when(i == nsteps - 1)
        def _exchange_and_merge():
            my_x = lax.axis_index("x")
            my_y = lax.axis_index("y")
            my_z = lax.axis_index("z")
            partner = (my_x, 1 - my_y, my_z)

            acc_tx[...] = acc[...].astype(jnp.bfloat16)
            stats_tx[0] = m[...]
            stats_tx[1] = l[...]
            rdma_o = pltpu.make_async_remote_copy(
                src_ref=acc_tx, dst_ref=acc_rx,
                send_sem=o_send, recv_sem=o_recv,
                device_id=partner, device_id_type=pl.DeviceIdType.MESH)
            rdma_s = pltpu.make_async_remote_copy(
                src_ref=stats_tx, dst_ref=stats_rx,
                send_sem=s_send, recv_sem=s_recv,
                device_id=partner, device_id_type=pl.DeviceIdType.MESH)
            rdma_o.start()
            rdma_s.start()
            rdma_o.wait()
            rdma_s.wait()

            m_l = stats_tx[0]
            l_l = stats_tx[1]
            m_r = stats_rx[0]
            l_r = stats_rx[1]
            m_n = jnp.maximum(m_l, m_r)
            a_l = jnp.exp(m_l - m_n)
            a_r = jnp.exp(m_r - m_n)
            l_n = l_l * a_l + l_r * a_r
            for hh in range(H):
                a_lh = a_l[:, hh]
                a_rh = a_r[:, hh]
                o_h = (acc[hh] * a_lh
                       + acc_rx[hh].astype(jnp.float32) * a_rh) / l_n[:, hh]
                o_ref[:, :, hh * D:(hh + 1) * D] = o_h

    out = pl.pallas_call(
        body,
        grid=(nsteps,),
        in_specs=[
            pl.BlockSpec((B, SQ, H * D), lambda i: (0, 0, 0)),
            pl.BlockSpec((B, BK, H * D), lambda i: (0, i, 0)),
            pl.BlockSpec((B, BK, H * D), lambda i: (0, i, 0)),
        ],
        out_specs=pl.BlockSpec((B, SQ, H * D), lambda i: (0, 0, 0)),
        out_shape=jax.ShapeDtypeStruct((B, SQ, H * D), jnp.float32),
        scratch_shapes=[
            pltpu.VMEM((H, B, SQ, D), jnp.float32),
            pltpu.VMEM((B, H, SQ, 1), jnp.float32),
            pltpu.VMEM((B, H, SQ, 1), jnp.float32),
            pltpu.VMEM((H, B, SQ, D), jnp.bfloat16),
            pltpu.VMEM((H, B, SQ, D), jnp.bfloat16),
            pltpu.VMEM((2, B, H, SQ, 1), jnp.float32),
            pltpu.VMEM((2, B, H, SQ, 1), jnp.float32),
            pltpu.SemaphoreType.DMA,
            pltpu.SemaphoreType.DMA,
            pltpu.SemaphoreType.DMA,
            pltpu.SemaphoreType.DMA,
        ],
        compiler_params=pltpu.CompilerParams(
            dimension_semantics=("arbitrary",),
            vmem_limit_bytes=100 * 1024 * 1024),
    )(Q, K, V)
    return out.reshape(B, SQ, H, D)


# device time: 61789 ns/iter; 4.1912x vs baseline; 4.1912x over previous
import jax
import jax.numpy as jnp
from jax import lax
from jax.experimental import pallas as pl
from jax.experimental.pallas import tpu as pltpu

B, SQ, H, D = 4, 32, 8, 128


def kernel(Q, K, V):
    skv = K.shape[1]
    scale = D ** -0.5

    def body(q_ref, k_hbm, v_hbm, o_ref,
             kb, vb, acc, acc_rx, stats_tx, stats_rx,
             k_sems, v_sems, o_send, o_recv, s_send, s_recv):
        h = pl.program_id(0)
        slot = lax.rem(h, 2)

        def kv_copy(hh, sl):
            ck = pltpu.make_async_copy(
                k_hbm.at[:, :, hh, :], kb.at[sl], k_sems.at[sl])
            cv = pltpu.make_async_copy(
                v_hbm.at[:, :, hh, :], vb.at[sl], v_sems.at[sl])
            return ck, cv

        def partner_rdma(hh):
            my_x = lax.axis_index("x")
            my_y = lax.axis_index("y")
            my_z = lax.axis_index("z")
            partner = (my_x, 1 - my_y, my_z)
            rdma_o = pltpu.make_async_remote_copy(
                src_ref=acc.at[hh], dst_ref=acc_rx.at[hh],
                send_sem=o_send.at[hh], recv_sem=o_recv.at[hh],
                device_id=partner, device_id_type=pl.DeviceIdType.MESH)
            rdma_s = pltpu.make_async_remote_copy(
                src_ref=stats_tx.at[hh], dst_ref=stats_rx.at[hh],
                send_sem=s_send.at[hh], recv_sem=s_recv.at[hh],
                device_id=partner, device_id_type=pl.DeviceIdType.MESH)
            return rdma_o, rdma_s

        @pl.when(h == 0)
        def _first_fetch():
            ck, cv = kv_copy(0, 0)
            ck.start()
            cv.start()

        @pl.when(h + 1 < H)
        def _prefetch_next():
            ck, cv = kv_copy(h + 1, lax.rem(h + 1, 2))
            ck.start()
            cv.start()

        ck, cv = kv_copy(h, slot)
        ck.wait()
        cv.wait()

        for bb in range(B):
            q = q_ref[bb, :, h, :].astype(jnp.bfloat16)
            k = kb[slot, bb].astype(jnp.bfloat16)
            v = vb[slot, bb].astype(jnp.bfloat16)
            s = lax.dot_general(
                q, k, (((1,), (1,)), ((), ())),
                preferred_element_type=jnp.float32) * scale
            m_c = jnp.max(s, axis=-1, keepdims=True)
            p = jnp.exp(s - m_c)
            l_c = jnp.sum(p, axis=-1, keepdims=True)
            pv = lax.dot_general(
                p.astype(jnp.bfloat16), v, (((1,), (0,)), ((), ())),
                preferred_element_type=jnp.float32)
            acc[h, bb] = pv
            stats_tx[h, 0, bb] = m_c
            stats_tx[h, 1, bb] = l_c

        rdma_o, rdma_s = partner_rdma(h)
        rdma_o.start()
        rdma_s.start()

        @pl.when(h == H - 1)
        def _merge():
            for hh in range(H):
                w_o, w_s = partner_rdma(hh)
                w_o.wait()
                w_s.wait()
            m_l = stats_tx[:, 0]
            l_l = stats_tx[:, 1]
            m_r = stats_rx[:, 0]
            l_r = stats_rx[:, 1]
            m_n = jnp.maximum(m_l, m_r)
            a_l = jnp.exp(m_l - m_n)
            a_r = jnp.exp(m_r - m_n)
            l_n = l_l * a_l + l_r * a_r
            o = (acc[...] * a_l + acc_rx[...] * a_r) / l_n
            for hh in range(H):
                o_ref[:, :, hh, :] = o[hh]

    return pl.pallas_call(
        body,
        grid=(H,),
        in_specs=[
            pl.BlockSpec((B, SQ, H, D), lambda h: (0, 0, 0, 0)),
            pl.BlockSpec(memory_space=pltpu.MemorySpace.HBM),
            pl.BlockSpec(memory_space=pltpu.MemorySpace.HBM),
        ],
        out_specs=pl.BlockSpec((B, SQ, H, D), lambda h: (0, 0, 0, 0)),
        out_shape=jax.ShapeDtypeStruct((B, SQ, H, D), jnp.float32),
        scratch_shapes=[
            pltpu.VMEM((2, B, skv, D), jnp.float32),
            pltpu.VMEM((2, B, skv, D), jnp.float32),
            pltpu.VMEM((H, B, SQ, D), jnp.float32),
            pltpu.VMEM((H, B, SQ, D), jnp.float32),
            pltpu.VMEM((H, 2, B, SQ, 1), jnp.float32),
            pltpu.VMEM((H, 2, B, SQ, 1), jnp.float32),
            pltpu.SemaphoreType.DMA((2,)),
            pltpu.SemaphoreType.DMA((2,)),
            pltpu.SemaphoreType.DMA((H,)),
            pltpu.SemaphoreType.DMA((H,)),
            pltpu.SemaphoreType.DMA((H,)),
            pltpu.SemaphoreType.DMA((H,)),
        ],
        compiler_params=pltpu.CompilerParams(
            dimension_semantics=("arbitrary",),
            vmem_limit_bytes=100 * 1024 * 1024),
    )(Q, K, V)
